# baseline (speedup 1.0000x reference)
import functools

import jax
import jax.numpy as jnp
from jax import lax
from jax.experimental import pallas as pl
from jax.experimental.pallas import tpu as pltpu
from jax.experimental.pallas import tpu_sc as plsc

N = 1_000_000
NW = 32
PER_W = 31_232
MAIN = NW * PER_W
TAIL = N - MAIN
KCH = 4
CHUNK = PER_W // KCH
UNROLL = 4
IN_IT = CHUNK // 16
NBINS = 64
ROW = 256
LN2 = 0.69314718055994530942
SQRT2 = 1.4142135623730951


def _vlog(x):
    bits = plsc.bitcast(x, jnp.int32)
    e = (bits >> 23) - 127
    m = plsc.bitcast((bits & 0x007FFFFF) | 0x3F800000, jnp.float32)
    ef = e.astype(jnp.float32)
    s = (m - 1.0) / (m + 1.0)
    t = s * s
    poly = 1.0 / 3.0 + t * (1.0 / 5.0 + t * (1.0 / 7.0 + t * (1.0 / 9.0)))
    return ef * LN2 + 2.0 * s * (1.0 + t * poly)


def _sc_body(conf_hbm, lf_hbm, tgt_hbm, out_hbm,
             cbufA, b0A, b1A, b2A, tbufA, cbufB, b0B, b1B, b2B, tbufB,
             ec, e0, e1, e2, et, hs, hr, acc, prow, semA, semB):
    nc = 2
    wid = lax.axis_index("s") * nc + lax.axis_index("c")
    base = wid * PER_W
    lane = lax.iota(jnp.int32, 16)
    zero16 = jnp.zeros((16,), jnp.float32)

    for i in range(NBINS):
        hs[pl.ds(i * 16, 16)] = zero16
        hr[pl.ds(i * 16, 16)] = zero16

    def sample16(i, cb, lb0, lb1, lb2, tb):
        conf = cb[pl.ds(i * 16, 16)]
        tgt = tb[pl.ds(i * 16, 16)]
        l0 = lb0[pl.ds(i * 16, 16)]
        l1 = lb1[pl.ds(i * 16, 16)]
        l2 = lb2[pl.ds(i * 16, 16)]
        pred = jnp.where(l2 > jnp.maximum(l0, l1), 2, jnp.where(l1 > l0, 1, 0))
        corr = (pred == tgt).astype(jnp.float32)
        p = jnp.maximum(conf, 1e-12)
        q = jnp.where(corr > 0.5, p, 1.0 - p)
        y = conf * 50.0
        iy = y.astype(jnp.int32)
        j = iy + (y > iy.astype(jnp.float32)).astype(jnp.int32) - 1
        valid = j >= 0
        hidx = lane * NBINS + jnp.maximum(j, 0)
        plsc.addupdate_scatter(hs, [hidx], conf, mask=valid)
        plsc.addupdate_scatter(hr, [hidx], corr + 4096.0, mask=valid)
        return -_vlog(q)

    bufs = [(cbufA, b0A, b1A, b2A, tbufA, semA), (cbufB, b0B, b1B, b2B, tbufB, semB)]

    def start_chunk(g):
        cb, x0, x1, x2, tb, sem = bufs[g % 2]
        off = base + g * CHUNK
        copies = (
            pltpu.async_copy(conf_hbm.at[0, pl.ds(off, CHUNK)], cb, sem),
            pltpu.async_copy(lf_hbm.at[pl.ds(off, CHUNK)], x0, sem),
            pltpu.async_copy(lf_hbm.at[pl.ds(N + off, CHUNK)], x1, sem),
            pltpu.async_copy(lf_hbm.at[pl.ds(2 * N + off, CHUNK)], x2, sem),
            pltpu.async_copy(tgt_hbm.at[pl.ds(off, CHUNK)], tb, sem),
        )
        return copies

    def compute_chunk(copies, g, a):
        cb, x0, x1, x2, tb, sem = bufs[g % 2]
        for cp in copies:
            cp.wait()

        def inner(i, a2):
            for u in range(UNROLL):
                a2 = a2 + sample16(i * UNROLL + u, cb, x0, x1, x2, tb)
            return a2

        return lax.fori_loop(0, IN_IT // UNROLL, inner, a)

    acc_main = zero16
    pending = start_chunk(0)
    for g in range(KCH):
        nxt = start_chunk(g + 1) if g + 1 < KCH else None
        acc_main = compute_chunk(pending, g, acc_main)
        pending = nxt
    acc[...] = zero16

    @pl.when(wid == 0)
    def _tail():
        pltpu.sync_copy(conf_hbm.at[0, pl.ds(MAIN, TAIL)], ec)
        pltpu.sync_copy(lf_hbm.at[pl.ds(MAIN, TAIL)], e0)
        pltpu.sync_copy(lf_hbm.at[pl.ds(N + MAIN, TAIL)], e1)
        pltpu.sync_copy(lf_hbm.at[pl.ds(2 * N + MAIN, TAIL)], e2)
        pltpu.sync_copy(tgt_hbm.at[pl.ds(MAIN, TAIL)], et)
        a2 = zero16
        for i in range(TAIL // 16):
            a2 = a2 + sample16(i, ec, e0, e1, e2, et)
        acc[...] = a2

    for g in range(4):
        vc = zero16
        vs = zero16
        vr = zero16
        for l in range(16):
            o = l * NBINS + g * 16
            vs = vs + hs[pl.ds(o, 16)]
            packed = hr[pl.ds(o, 16)]
            cnt = (packed * (1.0 / 4096.0)).astype(jnp.int32).astype(jnp.float32)
            vc = vc + cnt
            vr = vr + (packed - cnt * 4096.0)
        prow[pl.ds(g * 16, 16)] = vc
        prow[pl.ds(64 + g * 16, 16)] = vs
        prow[pl.ds(128 + g * 16, 16)] = vr
    prow[pl.ds(192, 16)] = acc_main + acc[...]
    prow[pl.ds(208, 16)] = zero16
    prow[pl.ds(224, 16)] = zero16
    prow[pl.ds(240, 16)] = zero16
    pltpu.sync_copy(prow, out_hbm.at[pl.ds(wid * ROW, ROW)])


_sc_hist = functools.partial(
    pl.kernel,
    out_type=jax.ShapeDtypeStruct((NW * ROW,), jnp.float32),
    mesh=plsc.VectorSubcoreMesh(core_axis_name="c", subcore_axis_name="s"),
    compiler_params=pltpu.CompilerParams(needs_layout_passes=False),
    scratch_types=[
        pltpu.VMEM((CHUNK,), jnp.float32),
        pltpu.VMEM((CHUNK,), jnp.float32),
        pltpu.VMEM((CHUNK,), jnp.float32),
        pltpu.VMEM((CHUNK,), jnp.float32),
        pltpu.VMEM((CHUNK,), jnp.int32),
        pltpu.VMEM((CHUNK,), jnp.float32),
        pltpu.VMEM((CHUNK,), jnp.float32),
        pltpu.VMEM((CHUNK,), jnp.float32),
        pltpu.VMEM((CHUNK,), jnp.float32),
        pltpu.VMEM((CHUNK,), jnp.int32),
        pltpu.VMEM((TAIL,), jnp.float32),
        pltpu.VMEM((TAIL,), jnp.float32),
        pltpu.VMEM((TAIL,), jnp.float32),
        pltpu.VMEM((TAIL,), jnp.float32),
        pltpu.VMEM((TAIL,), jnp.int32),
        pltpu.VMEM((16 * NBINS,), jnp.float32),
        pltpu.VMEM((16 * NBINS,), jnp.float32),
        pltpu.VMEM((16,), jnp.float32),
        pltpu.VMEM((ROW,), jnp.float32),
        pltpu.SemaphoreType.DMA,
        pltpu.SemaphoreType.DMA,
    ],
)(_sc_body)


def _combine(x_ref, t_ref, b_ref, e_ref):
    x = x_ref[...]
    nf = jnp.float32(N)
    cnt = jnp.sum(x[:, 0:64], axis=0, keepdims=True)
    sconf = jnp.sum(x[:, 64:128], axis=0, keepdims=True)
    scorr = jnp.sum(x[:, 128:192], axis=0, keepdims=True)
    bce = jnp.sum(x[:, 192:208]) / nf
    safe = jnp.maximum(cnt, 1.0)
    term = jnp.where(cnt > 0, (cnt / nf) * jnp.abs(scorr / safe - sconf / safe), 0.0)
    ece = jnp.sum(term)
    t_ref[0, 0] = bce + ece
    b_ref[0, 0] = bce
    e_ref[0, 0] = ece


def kernel(confidence, direction_logits, targets):
    conf = confidence.T
    lflat = direction_logits.T.reshape(3 * N)
    x = _sc_hist(conf, lflat, targets).reshape(NW, ROW)
    nf = jnp.float32(N)
    cnt = jnp.sum(x[:, 0:64], axis=0)
    sconf = jnp.sum(x[:, 64:128], axis=0)
    scorr = jnp.sum(x[:, 128:192], axis=0)
    bce = jnp.sum(x[:, 192:208]) / nf
    safe = jnp.maximum(cnt, 1.0)
    term = jnp.where(cnt > 0, (cnt / nf) * jnp.abs(scorr / safe - sconf / safe), 0.0)
    ece = jnp.sum(term)
    return (bce + ece, bce, ece)


# speedup vs baseline: 1.0296x; 1.0296x over previous
import functools

import jax
import jax.numpy as jnp
from jax import lax
from jax.experimental import pallas as pl
from jax.experimental.pallas import tpu as pltpu
from jax.experimental.pallas import tpu_sc as plsc

N = 1_000_000
NW = 32
PER_W = 31_232
MAIN = NW * PER_W
TAIL = N - MAIN
KCH = 4
CHUNK = PER_W // KCH
UNROLL = 4
IN_IT = CHUNK // 16
NBINS = 64
ROW = 256
LN2 = 0.69314718055994530942
SQRT2 = 1.4142135623730951


def _vlog(x):
    bits = plsc.bitcast(x, jnp.int32)
    e = (bits >> 23) - 127
    m = plsc.bitcast((bits & 0x007FFFFF) | 0x3F800000, jnp.float32)
    ef = e.astype(jnp.float32)
    s = (m - 1.0) / (m + 1.0)
    t = s * s
    poly = 1.0 / 3.0 + t * (1.0 / 5.0 + t * (1.0 / 7.0 + t * (1.0 / 9.0)))
    return ef * LN2 + 2.0 * s * (1.0 + t * poly)


def _sc_body(conf_hbm, lf_hbm, tgt_hbm, out_hbm,
             cbufA, b0A, b1A, b2A, tbufA, cbufB, b0B, b1B, b2B, tbufB,
             ec, e0, e1, e2, et, hs, hr, acc, prow, semA, semB):
    nc = 2
    wid = lax.axis_index("s") * nc + lax.axis_index("c")
    base = wid * PER_W
    lane = lax.iota(jnp.int32, 16)
    zero16 = jnp.zeros((16,), jnp.float32)

    for i in range(NBINS):
        hs[pl.ds(i * 16, 16)] = zero16
        hr[pl.ds(i * 16, 16)] = zero16

    def sample16(i, cb, lb0, lb1, lb2, tb):
        conf = cb[pl.ds(i * 16, 16)]
        tgt = tb[pl.ds(i * 16, 16)]
        l0 = lb0[pl.ds(i * 16, 16)]
        l1 = lb1[pl.ds(i * 16, 16)]
        l2 = lb2[pl.ds(i * 16, 16)]
        pred = jnp.where(l2 > jnp.maximum(l0, l1), 2, jnp.where(l1 > l0, 1, 0))
        corr = (pred == tgt).astype(jnp.float32)
        p = jnp.maximum(conf, 1e-12)
        q = jnp.where(corr > 0.5, p, 1.0 - p)
        y = conf * 50.0
        iy = y.astype(jnp.int32)
        j = iy + (y > iy.astype(jnp.float32)).astype(jnp.int32) - 1
        valid = j >= 0
        hidx = lane * NBINS + jnp.maximum(j, 0)
        plsc.addupdate_scatter(hs, [hidx], conf, mask=valid)
        plsc.addupdate_scatter(hr, [hidx], corr + 4096.0, mask=valid)
        return -_vlog(q)

    bufs = [(cbufA, b0A, b1A, b2A, tbufA, semA), (cbufB, b0B, b1B, b2B, tbufB, semB)]

    def start_chunk(g):
        cb, x0, x1, x2, tb, sem = bufs[g % 2]
        off = base + g * CHUNK
        copies = (
            pltpu.async_copy(conf_hbm.at[0, pl.ds(off, CHUNK)], cb, sem),
            pltpu.async_copy(lf_hbm.at[pl.ds(off, CHUNK)], x0, sem),
            pltpu.async_copy(lf_hbm.at[pl.ds(N + off, CHUNK)], x1, sem),
            pltpu.async_copy(lf_hbm.at[pl.ds(2 * N + off, CHUNK)], x2, sem),
            pltpu.async_copy(tgt_hbm.at[pl.ds(off, CHUNK)], tb, sem),
        )
        return copies

    def compute_chunk(copies, g, a):
        cb, x0, x1, x2, tb, sem = bufs[g % 2]
        for cp in copies:
            cp.wait()

        def inner(i, a2):
            for u in range(UNROLL):
                a2 = a2 + sample16(i * UNROLL + u, cb, x0, x1, x2, tb)
            return a2

        return lax.fori_loop(0, IN_IT // UNROLL, inner, a)

    acc_main = zero16
    pending = start_chunk(0)
    for g in range(KCH):
        nxt = start_chunk(g + 1) if g + 1 < KCH else None
        acc_main = compute_chunk(pending, g, acc_main)
        pending = nxt
    acc[...] = zero16

    @pl.when(wid == 0)
    def _tail():
        pltpu.sync_copy(conf_hbm.at[0, pl.ds(MAIN, TAIL)], ec)
        pltpu.sync_copy(lf_hbm.at[pl.ds(MAIN, TAIL)], e0)
        pltpu.sync_copy(lf_hbm.at[pl.ds(N + MAIN, TAIL)], e1)
        pltpu.sync_copy(lf_hbm.at[pl.ds(2 * N + MAIN, TAIL)], e2)
        pltpu.sync_copy(tgt_hbm.at[pl.ds(MAIN, TAIL)], et)
        a2 = zero16
        for i in range(TAIL // 16):
            a2 = a2 + sample16(i, ec, e0, e1, e2, et)
        acc[...] = a2

    for g in range(4):
        vc = zero16
        vs = zero16
        vr = zero16
        for l in range(16):
            o = l * NBINS + g * 16
            vs = vs + hs[pl.ds(o, 16)]
            packed = hr[pl.ds(o, 16)]
            cnt = (packed * (1.0 / 4096.0)).astype(jnp.int32).astype(jnp.float32)
            vc = vc + cnt
            vr = vr + (packed - cnt * 4096.0)
        prow[pl.ds(g * 16, 16)] = vc
        prow[pl.ds(64 + g * 16, 16)] = vs
        prow[pl.ds(128 + g * 16, 16)] = vr
    prow[pl.ds(192, 16)] = acc_main + acc[...]
    prow[pl.ds(208, 16)] = zero16
    prow[pl.ds(224, 16)] = zero16
    prow[pl.ds(240, 16)] = zero16
    pltpu.sync_copy(prow, out_hbm.at[pl.ds(wid * ROW, ROW)])


_sc_hist = functools.partial(
    pl.kernel,
    out_type=jax.ShapeDtypeStruct((NW * ROW,), jnp.float32),
    mesh=plsc.VectorSubcoreMesh(core_axis_name="c", subcore_axis_name="s"),
    compiler_params=pltpu.CompilerParams(needs_layout_passes=False),
    scratch_types=[
        pltpu.VMEM((CHUNK,), jnp.float32),
        pltpu.VMEM((CHUNK,), jnp.float32),
        pltpu.VMEM((CHUNK,), jnp.float32),
        pltpu.VMEM((CHUNK,), jnp.float32),
        pltpu.VMEM((CHUNK,), jnp.int32),
        pltpu.VMEM((CHUNK,), jnp.float32),
        pltpu.VMEM((CHUNK,), jnp.float32),
        pltpu.VMEM((CHUNK,), jnp.float32),
        pltpu.VMEM((CHUNK,), jnp.float32),
        pltpu.VMEM((CHUNK,), jnp.int32),
        pltpu.VMEM((TAIL,), jnp.float32),
        pltpu.VMEM((TAIL,), jnp.float32),
        pltpu.VMEM((TAIL,), jnp.float32),
        pltpu.VMEM((TAIL,), jnp.float32),
        pltpu.VMEM((TAIL,), jnp.int32),
        pltpu.VMEM((16 * NBINS,), jnp.float32),
        pltpu.VMEM((16 * NBINS,), jnp.float32),
        pltpu.VMEM((16,), jnp.float32),
        pltpu.VMEM((ROW,), jnp.float32),
        pltpu.SemaphoreType.DMA,
        pltpu.SemaphoreType.DMA,
    ],
)(_sc_body)


def _combine(x_ref, t_ref, b_ref, e_ref):
    x = x_ref[...]
    nf = jnp.float32(N)
    cnt = jnp.sum(x[:, 0:64], axis=0, keepdims=True)
    sconf = jnp.sum(x[:, 64:128], axis=0, keepdims=True)
    scorr = jnp.sum(x[:, 128:192], axis=0, keepdims=True)
    bce = jnp.sum(x[:, 192:208]) / nf
    safe = jnp.maximum(cnt, 1.0)
    term = jnp.where(cnt > 0, (cnt / nf) * jnp.abs(scorr / safe - sconf / safe), 0.0)
    ece = jnp.sum(term)
    t_ref[0, 0] = bce + ece
    b_ref[0, 0] = bce
    e_ref[0, 0] = ece


def kernel(confidence, direction_logits, targets):
    conf = confidence.T
    lflat = direction_logits.T.reshape(3 * N)
    partial = _sc_hist(conf, lflat, targets)
    total, bce, ece = pl.pallas_call(
        _combine,
        out_shape=(
            jax.ShapeDtypeStruct((1, 1), jnp.float32),
            jax.ShapeDtypeStruct((1, 1), jnp.float32),
            jax.ShapeDtypeStruct((1, 1), jnp.float32),
        ),
        out_specs=(
            pl.BlockSpec(memory_space=pltpu.SMEM),
            pl.BlockSpec(memory_space=pltpu.SMEM),
            pl.BlockSpec(memory_space=pltpu.SMEM),
        ),
    )(partial.reshape(NW, ROW))
    return (total[0, 0], bce[0, 0], ece[0, 0])


# speedup vs baseline: 1.0530x; 1.0227x over previous
import functools

import jax
import jax.numpy as jnp
from jax import lax
from jax.experimental import pallas as pl
from jax.experimental.pallas import tpu as pltpu
from jax.experimental.pallas import tpu_sc as plsc

N = 1_000_000
NW = 32
HALF_A = 499_968
HALF_B = N - HALF_A
PER_W = 15_616
CHUNK = 7808
KCH = 2
UNROLL = 4
IN_IT = CHUNK // 16
NBINS = 64
ROW = 256
LN2 = 0.69314718055994530942


def _vlog(x):
    bits = plsc.bitcast(x, jnp.int32)
    e = (bits >> 23) - 127
    m = plsc.bitcast((bits & 0x007FFFFF) | 0x3F800000, jnp.float32)
    ef = e.astype(jnp.float32)
    s = (m - 1.0) / (m + 1.0)
    t = s * s
    poly = 1.0 / 3.0 + t * (1.0 / 5.0 + t * (1.0 / 7.0 + t * (1.0 / 9.0)))
    return ef * LN2 + 2.0 * s * (1.0 + t * poly)


def _make_sc_body(sbase, nh):
    main = NW * PER_W
    tail = nh - main
    assert tail % 16 == 0 and (sbase + main) % 128 == 0

    def body(conf_hbm, lf_hbm, tgt_hbm, out_hbm,
             cbufA, b0A, b1A, b2A, tbufA, cbufB, b0B, b1B, b2B, tbufB,
             ec, e0, e1, e2, et, hs, hr, acc, prow, semA, semB):
        nc = 2
        wid = lax.axis_index("s") * nc + lax.axis_index("c")
        base = wid * PER_W
        lane = lax.iota(jnp.int32, 16)
        zero16 = jnp.zeros((16,), jnp.float32)

        for i in range(NBINS):
            hs[pl.ds(i * 16, 16)] = zero16
            hr[pl.ds(i * 16, 16)] = zero16

        def sample16(i, cb, lb0, lb1, lb2, tb):
            conf = cb[pl.ds(i * 16, 16)]
            tgt = tb[pl.ds(i * 16, 16)]
            l0 = lb0[pl.ds(i * 16, 16)]
            l1 = lb1[pl.ds(i * 16, 16)]
            l2 = lb2[pl.ds(i * 16, 16)]
            pred = jnp.where(l2 > jnp.maximum(l0, l1), 2, jnp.where(l1 > l0, 1, 0))
            corr = (pred == tgt).astype(jnp.float32)
            p = jnp.maximum(conf, 1e-12)
            q = jnp.where(corr > 0.5, p, 1.0 - p)
            y = conf * 50.0
            iy = y.astype(jnp.int32)
            j = iy + (y > iy.astype(jnp.float32)).astype(jnp.int32) - 1
            valid = j >= 0
            hidx = lane * NBINS + jnp.maximum(j, 0)
            plsc.addupdate_scatter(hs, [hidx], conf, mask=valid)
            plsc.addupdate_scatter(hr, [hidx], corr + 4096.0, mask=valid)
            return -_vlog(q)

        bufs = [(cbufA, b0A, b1A, b2A, tbufA, semA), (cbufB, b0B, b1B, b2B, tbufB, semB)]

        def start_chunk(g):
            cb, x0, x1, x2, tb, sem = bufs[g % 2]
            loc = base + g * CHUNK
            gl = sbase + loc
            return (
                pltpu.async_copy(conf_hbm.at[0, pl.ds(gl, CHUNK)], cb, sem),
                pltpu.async_copy(lf_hbm.at[pl.ds(loc, CHUNK)], x0, sem),
                pltpu.async_copy(lf_hbm.at[pl.ds(nh + loc, CHUNK)], x1, sem),
                pltpu.async_copy(lf_hbm.at[pl.ds(2 * nh + loc, CHUNK)], x2, sem),
                pltpu.async_copy(tgt_hbm.at[pl.ds(gl, CHUNK)], tb, sem),
            )

        def compute_chunk(copies, g, a):
            cb, x0, x1, x2, tb, sem = bufs[g % 2]
            for cp in copies:
                cp.wait()

            def inner(i, a2):
                for u in range(UNROLL):
                    a2 = a2 + sample16(i * UNROLL + u, cb, x0, x1, x2, tb)
                return a2

            return lax.fori_loop(0, IN_IT // UNROLL, inner, a)

        acc_main = zero16
        pending = start_chunk(0)
        for g in range(KCH):
            nxt = start_chunk(g + 1) if g + 1 < KCH else None
            acc_main = compute_chunk(pending, g, acc_main)
            pending = nxt
        acc[...] = zero16

        @pl.when(wid == 0)
        def _tail():
            pltpu.sync_copy(conf_hbm.at[0, pl.ds(sbase + main, tail)], ec)
            pltpu.sync_copy(lf_hbm.at[pl.ds(main, tail)], e0)
            pltpu.sync_copy(lf_hbm.at[pl.ds(nh + main, tail)], e1)
            pltpu.sync_copy(lf_hbm.at[pl.ds(2 * nh + main, tail)], e2)
            pltpu.sync_copy(tgt_hbm.at[pl.ds(sbase + main, tail)], et)
            a2 = zero16
            for i in range(tail // 16):
                a2 = a2 + sample16(i, ec, e0, e1, e2, et)
            acc[...] = a2

        for g in range(4):
            vc = zero16
            vs = zero16
            vr = zero16
            for l in range(16):
                o = l * NBINS + g * 16
                vs = vs + hs[pl.ds(o, 16)]
                packed = hr[pl.ds(o, 16)]
                cnt = (packed * (1.0 / 4096.0)).astype(jnp.int32).astype(jnp.float32)
                vc = vc + cnt
                vr = vr + (packed - cnt * 4096.0)
            prow[pl.ds(g * 16, 16)] = vc
            prow[pl.ds(64 + g * 16, 16)] = vs
            prow[pl.ds(128 + g * 16, 16)] = vr
        prow[pl.ds(192, 16)] = acc_main + acc[...]
        prow[pl.ds(208, 16)] = zero16
        prow[pl.ds(224, 16)] = zero16
        prow[pl.ds(240, 16)] = zero16
        pltpu.sync_copy(prow, out_hbm.at[pl.ds(wid * ROW, ROW)])

    return body


def _make_sc(sbase, nh):
    tail = nh - NW * PER_W
    return functools.partial(
        pl.kernel,
        out_type=jax.ShapeDtypeStruct((NW * ROW,), jnp.float32),
        mesh=plsc.VectorSubcoreMesh(core_axis_name="c", subcore_axis_name="s"),
        compiler_params=pltpu.CompilerParams(needs_layout_passes=False),
        scratch_types=[
            pltpu.VMEM((CHUNK,), jnp.float32),
            pltpu.VMEM((CHUNK,), jnp.float32),
            pltpu.VMEM((CHUNK,), jnp.float32),
            pltpu.VMEM((CHUNK,), jnp.float32),
            pltpu.VMEM((CHUNK,), jnp.int32),
            pltpu.VMEM((CHUNK,), jnp.float32),
            pltpu.VMEM((CHUNK,), jnp.float32),
            pltpu.VMEM((CHUNK,), jnp.float32),
            pltpu.VMEM((CHUNK,), jnp.float32),
            pltpu.VMEM((CHUNK,), jnp.int32),
            pltpu.VMEM((tail,), jnp.float32),
            pltpu.VMEM((tail,), jnp.float32),
            pltpu.VMEM((tail,), jnp.float32),
            pltpu.VMEM((tail,), jnp.float32),
            pltpu.VMEM((tail,), jnp.int32),
            pltpu.VMEM((16 * NBINS,), jnp.float32),
            pltpu.VMEM((16 * NBINS,), jnp.float32),
            pltpu.VMEM((16,), jnp.float32),
            pltpu.VMEM((ROW,), jnp.float32),
            pltpu.SemaphoreType.DMA,
            pltpu.SemaphoreType.DMA,
        ],
    )(_make_sc_body(sbase, nh))


_sc_a = _make_sc(0, HALF_A)
_sc_b = _make_sc(HALF_A, HALF_B)


def _combine(xa_ref, xb_ref, t_ref, b_ref, e_ref):
    x = xa_ref[...] + xb_ref[...]
    nf = jnp.float32(N)
    cnt = jnp.sum(x[:, 0:64], axis=0, keepdims=True)
    sconf = jnp.sum(x[:, 64:128], axis=0, keepdims=True)
    scorr = jnp.sum(x[:, 128:192], axis=0, keepdims=True)
    bce = jnp.sum(x[:, 192:208]) / nf
    safe = jnp.maximum(cnt, 1.0)
    term = jnp.where(cnt > 0, (cnt / nf) * jnp.abs(scorr / safe - sconf / safe), 0.0)
    ece = jnp.sum(term)
    t_ref[0, 0] = bce + ece
    b_ref[0, 0] = bce
    e_ref[0, 0] = ece


def kernel(confidence, direction_logits, targets):
    conf = confidence.T
    lt = direction_logits.T
    lfa = lt[:, :HALF_A].reshape(3 * HALF_A)
    ltb, _ = lax.optimization_barrier((lt, lfa))
    lfb = ltb[:, HALF_A:].reshape(3 * HALF_B)
    pa = _sc_a(conf, lfa, targets)
    pb = _sc_b(conf, lfb, targets)
    total, bce, ece = pl.pallas_call(
        _combine,
        out_shape=(
            jax.ShapeDtypeStruct((1, 1), jnp.float32),
            jax.ShapeDtypeStruct((1, 1), jnp.float32),
            jax.ShapeDtypeStruct((1, 1), jnp.float32),
        ),
        out_specs=(
            pl.BlockSpec(memory_space=pltpu.SMEM),
            pl.BlockSpec(memory_space=pltpu.SMEM),
            pl.BlockSpec(memory_space=pltpu.SMEM),
        ),
    )(pa.reshape(NW, ROW), pb.reshape(NW, ROW))
    return (total[0, 0], bce[0, 0], ece[0, 0])


# speedup vs baseline: 1.1104x; 1.0546x over previous
import functools

import jax
import jax.numpy as jnp
from jax import lax
from jax.experimental import pallas as pl
from jax.experimental.pallas import tpu as pltpu
from jax.experimental.pallas import tpu_sc as plsc

N = 1_000_000
NW = 32
HALF_A = 360_448
HALF_B = N - HALF_A
UNROLL = 4
NBINS = 64
ROW = 256
LN2 = 0.69314718055994530942


def _vlog(x):
    bits = plsc.bitcast(x, jnp.int32)
    e = (bits >> 23) - 127
    m = plsc.bitcast((bits & 0x007FFFFF) | 0x3F800000, jnp.float32)
    ef = e.astype(jnp.float32)
    s = (m - 1.0) / (m + 1.0)
    t = s * s
    poly = 1.0 / 3.0 + t * (1.0 / 5.0 + t * (1.0 / 7.0 + t * (1.0 / 9.0)))
    return ef * LN2 + 2.0 * s * (1.0 + t * poly)


def _make_sc_body(sbase, nh, per_w, chunk, kch):
    main = NW * per_w
    tail = nh - main
    in_it = chunk // 16
    assert per_w == kch * chunk and tail % 16 == 0
    assert sbase % 128 == 0 and chunk % 128 == 0 and (sbase + main) % 128 == 0
    assert in_it % UNROLL == 0

    def body(conf_hbm, lf_hbm, tgt_hbm, out_hbm,
             cbufA, b0A, b1A, b2A, tbufA, cbufB, b0B, b1B, b2B, tbufB,
             ec, e0, e1, e2, et, hs, hr, acc, prow, semA, semB):
        nc = 2
        wid = lax.axis_index("s") * nc + lax.axis_index("c")
        base = wid * per_w
        lane = lax.iota(jnp.int32, 16)
        zero16 = jnp.zeros((16,), jnp.float32)

        for i in range(NBINS):
            hs[pl.ds(i * 16, 16)] = zero16
            hr[pl.ds(i * 16, 16)] = zero16

        def sample16(i, cb, lb0, lb1, lb2, tb):
            conf = cb[pl.ds(i * 16, 16)]
            tgt = tb[pl.ds(i * 16, 16)]
            l0 = lb0[pl.ds(i * 16, 16)]
            l1 = lb1[pl.ds(i * 16, 16)]
            l2 = lb2[pl.ds(i * 16, 16)]
            pred = jnp.where(l2 > jnp.maximum(l0, l1), 2, jnp.where(l1 > l0, 1, 0))
            corr = (pred == tgt).astype(jnp.float32)
            p = jnp.maximum(conf, 1e-12)
            q = jnp.where(corr > 0.5, p, 1.0 - p)
            y = conf * 50.0
            iy = y.astype(jnp.int32)
            j = iy + (y > iy.astype(jnp.float32)).astype(jnp.int32) - 1
            valid = j >= 0
            hidx = lane * NBINS + jnp.maximum(j, 0)
            plsc.addupdate_scatter(hs, [hidx], conf, mask=valid)
            plsc.addupdate_scatter(hr, [hidx], corr + 4096.0, mask=valid)
            return -_vlog(q)

        bufs = [(cbufA, b0A, b1A, b2A, tbufA, semA), (cbufB, b0B, b1B, b2B, tbufB, semB)]

        def start_chunk(g):
            cb, x0, x1, x2, tb, sem = bufs[g % 2]
            loc = base + g * chunk
            gl = sbase + loc
            return (
                pltpu.async_copy(conf_hbm.at[0, pl.ds(gl, chunk)], cb, sem),
                pltpu.async_copy(lf_hbm.at[pl.ds(loc, chunk)], x0, sem),
                pltpu.async_copy(lf_hbm.at[pl.ds(nh + loc, chunk)], x1, sem),
                pltpu.async_copy(lf_hbm.at[pl.ds(2 * nh + loc, chunk)], x2, sem),
                pltpu.async_copy(tgt_hbm.at[pl.ds(gl, chunk)], tb, sem),
            )

        def compute_chunk(copies, g, a):
            cb, x0, x1, x2, tb, sem = bufs[g % 2]
            for cp in copies:
                cp.wait()

            def inner(i, a2):
                for u in range(UNROLL):
                    a2 = a2 + sample16(i * UNROLL + u, cb, x0, x1, x2, tb)
                return a2

            return lax.fori_loop(0, in_it // UNROLL, inner, a)

        acc_main = zero16
        pending = start_chunk(0)
        for g in range(kch):
            nxt = start_chunk(g + 1) if g + 1 < kch else None
            acc_main = compute_chunk(pending, g, acc_main)
            pending = nxt
        acc[...] = zero16

        if tail > 0:
            @pl.when(wid == 0)
            def _tail():
                pltpu.sync_copy(conf_hbm.at[0, pl.ds(sbase + main, tail)], ec)
                pltpu.sync_copy(lf_hbm.at[pl.ds(main, tail)], e0)
                pltpu.sync_copy(lf_hbm.at[pl.ds(nh + main, tail)], e1)
                pltpu.sync_copy(lf_hbm.at[pl.ds(2 * nh + main, tail)], e2)
                pltpu.sync_copy(tgt_hbm.at[pl.ds(sbase + main, tail)], et)
                a2 = zero16
                for i in range(tail // 16):
                    a2 = a2 + sample16(i, ec, e0, e1, e2, et)
                acc[...] = a2

        for g in range(4):
            vc = zero16
            vs = zero16
            vr = zero16
            for l in range(16):
                o = l * NBINS + g * 16
                vs = vs + hs[pl.ds(o, 16)]
                packed = hr[pl.ds(o, 16)]
                cnt = (packed * (1.0 / 4096.0)).astype(jnp.int32).astype(jnp.float32)
                vc = vc + cnt
                vr = vr + (packed - cnt * 4096.0)
            prow[pl.ds(g * 16, 16)] = vc
            prow[pl.ds(64 + g * 16, 16)] = vs
            prow[pl.ds(128 + g * 16, 16)] = vr
        prow[pl.ds(192, 16)] = acc_main + acc[...]
        prow[pl.ds(208, 16)] = zero16
        prow[pl.ds(224, 16)] = zero16
        prow[pl.ds(240, 16)] = zero16
        pltpu.sync_copy(prow, out_hbm.at[pl.ds(wid * ROW, ROW)])

    return body


def _make_sc(sbase, nh, per_w, chunk, kch):
    tail = max(nh - NW * per_w, 16)
    return functools.partial(
        pl.kernel,
        out_type=jax.ShapeDtypeStruct((NW * ROW,), jnp.float32),
        mesh=plsc.VectorSubcoreMesh(core_axis_name="c", subcore_axis_name="s"),
        compiler_params=pltpu.CompilerParams(needs_layout_passes=False),
        scratch_types=[
            pltpu.VMEM((chunk,), jnp.float32),
            pltpu.VMEM((chunk,), jnp.float32),
            pltpu.VMEM((chunk,), jnp.float32),
            pltpu.VMEM((chunk,), jnp.float32),
            pltpu.VMEM((chunk,), jnp.int32),
            pltpu.VMEM((chunk,), jnp.float32),
            pltpu.VMEM((chunk,), jnp.float32),
            pltpu.VMEM((chunk,), jnp.float32),
            pltpu.VMEM((chunk,), jnp.float32),
            pltpu.VMEM((chunk,), jnp.int32),
            pltpu.VMEM((tail,), jnp.float32),
            pltpu.VMEM((tail,), jnp.float32),
            pltpu.VMEM((tail,), jnp.float32),
            pltpu.VMEM((tail,), jnp.float32),
            pltpu.VMEM((tail,), jnp.int32),
            pltpu.VMEM((16 * NBINS,), jnp.float32),
            pltpu.VMEM((16 * NBINS,), jnp.float32),
            pltpu.VMEM((16,), jnp.float32),
            pltpu.VMEM((ROW,), jnp.float32),
            pltpu.SemaphoreType.DMA,
            pltpu.SemaphoreType.DMA,
        ],
    )(_make_sc_body(sbase, nh, per_w, chunk, kch))


_sc_a = _make_sc(0, HALF_A, 11_264, 5_632, 2)
_sc_b = _make_sc(HALF_A, HALF_B, 19_968, 4_992, 4)


def _combine(xa_ref, xb_ref, t_ref, b_ref, e_ref):
    x = xa_ref[...] + xb_ref[...]
    nf = jnp.float32(N)
    cnt = jnp.sum(x[:, 0:64], axis=0, keepdims=True)
    sconf = jnp.sum(x[:, 64:128], axis=0, keepdims=True)
    scorr = jnp.sum(x[:, 128:192], axis=0, keepdims=True)
    bce = jnp.sum(x[:, 192:208]) / nf
    safe = jnp.maximum(cnt, 1.0)
    term = jnp.where(cnt > 0, (cnt / nf) * jnp.abs(scorr / safe - sconf / safe), 0.0)
    ece = jnp.sum(term)
    t_ref[0, 0] = bce + ece
    b_ref[0, 0] = bce
    e_ref[0, 0] = ece


def kernel(confidence, direction_logits, targets):
    conf = confidence.T
    lt = direction_logits.T
    lfa = lt[:, :HALF_A].reshape(3 * HALF_A)
    ltb, _ = lax.optimization_barrier((lt, lfa))
    lfb = ltb[:, HALF_A:].reshape(3 * HALF_B)
    pa = _sc_a(conf, lfa, targets)
    pb = _sc_b(conf, lfb, targets)
    total, bce, ece = pl.pallas_call(
        _combine,
        out_shape=(
            jax.ShapeDtypeStruct((1, 1), jnp.float32),
            jax.ShapeDtypeStruct((1, 1), jnp.float32),
            jax.ShapeDtypeStruct((1, 1), jnp.float32),
        ),
        out_specs=(
            pl.BlockSpec(memory_space=pltpu.SMEM),
            pl.BlockSpec(memory_space=pltpu.SMEM),
            pl.BlockSpec(memory_space=pltpu.SMEM),
        ),
    )(pa.reshape(NW, ROW), pb.reshape(NW, ROW))
    return (total[0, 0], bce[0, 0], ece[0, 0])


# speedup vs baseline: 1.1442x; 1.0305x over previous
import functools

import jax
import jax.numpy as jnp
from jax import lax
from jax.experimental import pallas as pl
from jax.experimental.pallas import tpu as pltpu
from jax.experimental.pallas import tpu_sc as plsc

N = 1_000_000
NW = 32
HALF_A = 360_448
HALF_B = N - HALF_A
UNROLL = 4
NBINS = 64
ROW = 256
LN2 = 0.69314718055994530942


def _vlog(x):
    bits = plsc.bitcast(x, jnp.int32)
    e = (bits >> 23) - 127
    m = plsc.bitcast((bits & 0x007FFFFF) | 0x3F800000, jnp.float32)
    ef = e.astype(jnp.float32)
    s = (m - 1.0) / (m + 1.0)
    t = s * s
    poly = 1.0 / 3.0 + t * (1.0 / 5.0 + t * (1.0 / 7.0 + t * (1.0 / 9.0)))
    return ef * LN2 + 2.0 * s * (1.0 + t * poly)


def _make_sc_body(sbase, nh, per_w, chunk, kch):
    main = NW * per_w
    tail = nh - main
    in_it = chunk // 16
    assert per_w == kch * chunk and tail % 16 == 0
    assert sbase % 128 == 0 and chunk % 128 == 0 and (sbase + main) % 128 == 0
    assert in_it % UNROLL == 0

    def body(conf_hbm, lf_hbm, tgt_hbm, out_hbm,
             cbufA, b0A, b1A, b2A, tbufA, cbufB, b0B, b1B, b2B, tbufB,
             ec, e0, e1, e2, et, hs, hr, acc, prow, semA, semB, semT):
        nc = 2
        wid = lax.axis_index("s") * nc + lax.axis_index("c")
        base = wid * per_w
        lane = lax.iota(jnp.int32, 16)
        zero16 = jnp.zeros((16,), jnp.float32)

        for i in range(NBINS):
            hs[pl.ds(i * 16, 16)] = zero16
            hr[pl.ds(i * 16, 16)] = zero16

        def sample16(i, cb, lb0, lb1, lb2, tb):
            conf = cb[pl.ds(i * 16, 16)]
            tgt = tb[pl.ds(i * 16, 16)]
            l0 = lb0[pl.ds(i * 16, 16)]
            l1 = lb1[pl.ds(i * 16, 16)]
            l2 = lb2[pl.ds(i * 16, 16)]
            pred = jnp.where(l2 > jnp.maximum(l0, l1), 2, jnp.where(l1 > l0, 1, 0))
            corr = (pred == tgt).astype(jnp.float32)
            p = jnp.maximum(conf, 1e-12)
            q = jnp.where(corr > 0.5, p, 1.0 - p)
            y = conf * 50.0
            iy = y.astype(jnp.int32)
            j = iy + (y > iy.astype(jnp.float32)).astype(jnp.int32) - 1
            valid = j >= 0
            hidx = lane * NBINS + jnp.maximum(j, 0)
            plsc.addupdate_scatter(hs, [hidx], conf, mask=valid)
            plsc.addupdate_scatter(hr, [hidx], corr + 4096.0, mask=valid)
            return -_vlog(q)

        bufs = [(cbufA, b0A, b1A, b2A, tbufA, semA), (cbufB, b0B, b1B, b2B, tbufB, semB)]

        def start_chunk(g):
            cb, x0, x1, x2, tb, sem = bufs[g % 2]
            loc = base + g * chunk
            gl = sbase + loc
            return (
                pltpu.async_copy(conf_hbm.at[0, pl.ds(gl, chunk)], cb, sem),
                pltpu.async_copy(lf_hbm.at[pl.ds(loc, chunk)], x0, sem),
                pltpu.async_copy(lf_hbm.at[pl.ds(nh + loc, chunk)], x1, sem),
                pltpu.async_copy(lf_hbm.at[pl.ds(2 * nh + loc, chunk)], x2, sem),
                pltpu.async_copy(tgt_hbm.at[pl.ds(gl, chunk)], tb, sem),
            )

        def compute_chunk(copies, g, a):
            cb, x0, x1, x2, tb, sem = bufs[g % 2]
            for cp in copies:
                cp.wait()

            def inner(i, a2):
                for u in range(UNROLL):
                    a2 = a2 + sample16(i * UNROLL + u, cb, x0, x1, x2, tb)
                return a2

            return lax.fori_loop(0, in_it // UNROLL, inner, a)

        if tail > 0:
            tail_copies = (
                pltpu.make_async_copy(conf_hbm.at[0, pl.ds(sbase + main, tail)], ec, semT),
                pltpu.make_async_copy(lf_hbm.at[pl.ds(main, tail)], e0, semT),
                pltpu.make_async_copy(lf_hbm.at[pl.ds(nh + main, tail)], e1, semT),
                pltpu.make_async_copy(lf_hbm.at[pl.ds(2 * nh + main, tail)], e2, semT),
                pltpu.make_async_copy(tgt_hbm.at[pl.ds(sbase + main, tail)], et, semT),
            )

            @pl.when(wid == 0)
            def _tail_start():
                for cp in tail_copies:
                    cp.start()
        acc_main = zero16
        pending = start_chunk(0)
        for g in range(kch):
            nxt = start_chunk(g + 1) if g + 1 < kch else None
            acc_main = compute_chunk(pending, g, acc_main)
            pending = nxt
        acc[...] = zero16

        if tail > 0:
            @pl.when(wid == 0)
            def _tail():
                for cp in tail_copies:
                    cp.wait()
                a2 = zero16
                for i in range(tail // 16):
                    a2 = a2 + sample16(i, ec, e0, e1, e2, et)
                acc[...] = a2

        for g in range(4):
            vc = zero16
            vs = zero16
            vr = zero16
            for l in range(16):
                o = l * NBINS + g * 16
                vs = vs + hs[pl.ds(o, 16)]
                packed = hr[pl.ds(o, 16)]
                cnt = (packed * (1.0 / 4096.0)).astype(jnp.int32).astype(jnp.float32)
                vc = vc + cnt
                vr = vr + (packed - cnt * 4096.0)
            prow[pl.ds(g * 16, 16)] = vc
            prow[pl.ds(64 + g * 16, 16)] = vs
            prow[pl.ds(128 + g * 16, 16)] = vr
        prow[pl.ds(192, 16)] = acc_main + acc[...]
        prow[pl.ds(208, 16)] = zero16
        prow[pl.ds(224, 16)] = zero16
        prow[pl.ds(240, 16)] = zero16
        pltpu.sync_copy(prow, out_hbm.at[pl.ds(wid * ROW, ROW)])

    return body


def _make_sc(sbase, nh, per_w, chunk, kch):
    tail = max(nh - NW * per_w, 16)
    return functools.partial(
        pl.kernel,
        out_type=jax.ShapeDtypeStruct((NW * ROW,), jnp.float32),
        mesh=plsc.VectorSubcoreMesh(core_axis_name="c", subcore_axis_name="s"),
        compiler_params=pltpu.CompilerParams(needs_layout_passes=False),
        scratch_types=[
            pltpu.VMEM((chunk,), jnp.float32),
            pltpu.VMEM((chunk,), jnp.float32),
            pltpu.VMEM((chunk,), jnp.float32),
            pltpu.VMEM((chunk,), jnp.float32),
            pltpu.VMEM((chunk,), jnp.int32),
            pltpu.VMEM((chunk,), jnp.float32),
            pltpu.VMEM((chunk,), jnp.float32),
            pltpu.VMEM((chunk,), jnp.float32),
            pltpu.VMEM((chunk,), jnp.float32),
            pltpu.VMEM((chunk,), jnp.int32),
            pltpu.VMEM((tail,), jnp.float32),
            pltpu.VMEM((tail,), jnp.float32),
            pltpu.VMEM((tail,), jnp.float32),
            pltpu.VMEM((tail,), jnp.float32),
            pltpu.VMEM((tail,), jnp.int32),
            pltpu.VMEM((16 * NBINS,), jnp.float32),
            pltpu.VMEM((16 * NBINS,), jnp.float32),
            pltpu.VMEM((16,), jnp.float32),
            pltpu.VMEM((ROW,), jnp.float32),
            pltpu.SemaphoreType.DMA,
            pltpu.SemaphoreType.DMA,
            pltpu.SemaphoreType.DMA,
        ],
    )(_make_sc_body(sbase, nh, per_w, chunk, kch))


_sc_a = _make_sc(0, HALF_A, 11_264, 5_632, 2)
_sc_b = _make_sc(HALF_A, HALF_B, 19_968, 4_992, 4)


def _combine(xa_ref, xb_ref, t_ref, b_ref, e_ref):
    x = xa_ref[...] + xb_ref[...]
    nf = jnp.float32(N)
    cnt = jnp.sum(x[:, 0:64], axis=0, keepdims=True)
    sconf = jnp.sum(x[:, 64:128], axis=0, keepdims=True)
    scorr = jnp.sum(x[:, 128:192], axis=0, keepdims=True)
    bce = jnp.sum(x[:, 192:208]) / nf
    safe = jnp.maximum(cnt, 1.0)
    term = jnp.where(cnt > 0, (cnt / nf) * jnp.abs(scorr / safe - sconf / safe), 0.0)
    ece = jnp.sum(term)
    t_ref[0, 0] = bce + ece
    b_ref[0, 0] = bce
    e_ref[0, 0] = ece


def kernel(confidence, direction_logits, targets):
    conf = confidence.T
    lt = direction_logits.T
    lfa = lt[:, :HALF_A].reshape(3 * HALF_A)
    ltb, _ = lax.optimization_barrier((lt, lfa))
    lfb = ltb[:, HALF_A:].reshape(3 * HALF_B)
    pa = _sc_a(conf, lfa, targets)
    pb = _sc_b(conf, lfb, targets)
    total, bce, ece = pl.pallas_call(
        _combine,
        out_shape=(
            jax.ShapeDtypeStruct((1, 1), jnp.float32),
            jax.ShapeDtypeStruct((1, 1), jnp.float32),
            jax.ShapeDtypeStruct((1, 1), jnp.float32),
        ),
        out_specs=(
            pl.BlockSpec(memory_space=pltpu.SMEM),
            pl.BlockSpec(memory_space=pltpu.SMEM),
            pl.BlockSpec(memory_space=pltpu.SMEM),
        ),
    )(pa.reshape(NW, ROW), pb.reshape(NW, ROW))
    return (total[0, 0], bce[0, 0], ece[0, 0])
